# baseline (speedup 1.0000x reference)
import jax
import jax.numpy as jnp
from jax.experimental import pallas as pl
from jax.experimental.pallas import tpu as pltpu

_TILE = 1024


def _body(a_ref, b_ref, out_ref, rowmin_ref, gmax_ref):
    j = pl.program_id(0)
    a = a_ref[...]
    b = b_ref[...]
    s = 0.25 * jnp.sum(a * a, axis=1, keepdims=True)
    t = jnp.sum(b * b, axis=1)
    c = jax.lax.dot_general(
        a, b, dimension_numbers=(((1,), (1,)), ((), ())),
        preferred_element_type=jnp.float32)
    d2 = (s + t[None, :]) + c
    pos = jnp.where(d2 > 0.0, d2, jnp.inf)
    tmin = jnp.min(pos, axis=1, keepdims=True)
    tmax = jnp.maximum(jnp.max(d2), 0.0)

    @pl.when(j == 0)
    def _init():
        rowmin_ref[...] = tmin
        gmax_ref[0, 0] = tmax

    @pl.when(j != 0)
    def _acc():
        rowmin_ref[...] = jnp.minimum(rowmin_ref[...], tmin)
        gmax_ref[0, 0] = jnp.maximum(gmax_ref[0, 0], tmax)

    @pl.when(j == pl.num_programs(0) - 1)
    def _finish():
        rm = rowmin_ref[...]
        rm = jnp.where(rm == jnp.inf, gmax_ref[0, 0], rm)
        n = rm.shape[0]
        loss = -0.5 * jnp.sum(jnp.log(rm)) / n
        out_ref[...] = jnp.reshape(loss, (1, 1))


def kernel(anchor, positive, negative):
    n, d = anchor.shape
    am2 = -2.0 * anchor
    b = jnp.concatenate([anchor, positive, negative], axis=0)
    grid = b.shape[0] // _TILE

    out = pl.pallas_call(
        _body,
        grid=(grid,),
        in_specs=[
            pl.BlockSpec((n, d), lambda j: (0, 0)),
            pl.BlockSpec((_TILE, d), lambda j: (j, 0)),
        ],
        out_specs=pl.BlockSpec((1, 1), lambda j: (0, 0)),
        out_shape=jax.ShapeDtypeStruct((1, 1), jnp.float32),
        scratch_shapes=[
            pltpu.VMEM((n, 1), jnp.float32),
            pltpu.SMEM((1, 1), jnp.float32),
        ],
        compiler_params=pltpu.CompilerParams(
            dimension_semantics=("arbitrary",)),
    )(am2, b)
    return out[0, 0]


# speedup vs baseline: 1.1400x; 1.1400x over previous
import jax
import jax.numpy as jnp
from jax.experimental import pallas as pl
from jax.experimental.pallas import tpu as pltpu

_TILE = 1024


def _body(a_ref, b_ref, loss_ref, bad_ref, rowmin_ref, flag_ref):
    j = pl.program_id(0)
    n_anchor_tiles = a_ref.shape[0] // b_ref.shape[0]
    a = a_ref[...]
    b = b_ref[...]
    s = 0.25 * jnp.sum(a * a, axis=1, keepdims=True)
    t = jnp.sum(b * b, axis=1)
    c = jax.lax.dot_general(
        a, b, dimension_numbers=(((1,), (1,)), ((), ())),
        preferred_element_type=jnp.float32)
    d2 = (s + t[None, :]) + c

    @pl.when(j == 0)
    def _init_flag():
        flag_ref[0, 0] = 0.0

    @pl.when(j < n_anchor_tiles)
    def _anchor_tile():
        pos = jnp.where(d2 > 0.0, d2, jnp.inf)
        tmin = jnp.min(pos, axis=1, keepdims=True)

        @pl.when(j == 0)
        def _():
            rowmin_ref[...] = tmin

        @pl.when(j != 0)
        def _():
            rowmin_ref[...] = jnp.minimum(rowmin_ref[...], tmin)

    @pl.when(j >= n_anchor_tiles)
    def _plain_tile():
        tmin = jnp.min(d2, axis=1, keepdims=True)
        rowmin_ref[...] = jnp.minimum(rowmin_ref[...], tmin)
        flag_ref[0, 0] = jnp.maximum(
            flag_ref[0, 0],
            jnp.where(jnp.min(tmin) <= 0.0, 1.0, 0.0))

    @pl.when(j == pl.num_programs(0) - 1)
    def _finish():
        rm = rowmin_ref[...]
        n = rm.shape[0]
        loss = -0.5 * jnp.sum(jnp.log(rm)) / n
        loss_ref[...] = jnp.reshape(loss, (1, 1))
        bad = jnp.maximum(
            flag_ref[0, 0],
            jnp.where(jnp.max(rm) == jnp.inf, 1.0, 0.0))
        bad_ref[...] = jnp.reshape(bad, (1, 1))


def _exact_body(a_ref, b_ref, out_ref, rowmin_ref, gmax_ref):
    j = pl.program_id(0)
    a = a_ref[...]
    b = b_ref[...]
    s = 0.25 * jnp.sum(a * a, axis=1, keepdims=True)
    t = jnp.sum(b * b, axis=1)
    c = jax.lax.dot_general(
        a, b, dimension_numbers=(((1,), (1,)), ((), ())),
        preferred_element_type=jnp.float32)
    d2 = (s + t[None, :]) + c
    pos = jnp.where(d2 > 0.0, d2, jnp.inf)
    tmin = jnp.min(pos, axis=1, keepdims=True)
    tmax = jnp.maximum(jnp.max(d2), 0.0)

    @pl.when(j == 0)
    def _init():
        rowmin_ref[...] = tmin
        gmax_ref[0, 0] = tmax

    @pl.when(j != 0)
    def _acc():
        rowmin_ref[...] = jnp.minimum(rowmin_ref[...], tmin)
        gmax_ref[0, 0] = jnp.maximum(gmax_ref[0, 0], tmax)

    @pl.when(j == pl.num_programs(0) - 1)
    def _finish():
        rm = rowmin_ref[...]
        rm = jnp.where(rm == jnp.inf, gmax_ref[0, 0], rm)
        n = rm.shape[0]
        loss = -0.5 * jnp.sum(jnp.log(rm)) / n
        out_ref[...] = jnp.reshape(loss, (1, 1))


def kernel(anchor, positive, negative):
    n, d = anchor.shape
    am2 = -2.0 * anchor
    b = jnp.concatenate([anchor, positive, negative], axis=0)
    grid = b.shape[0] // _TILE

    loss_fast, bad = pl.pallas_call(
        _body,
        grid=(grid,),
        in_specs=[
            pl.BlockSpec((n, d), lambda j: (0, 0)),
            pl.BlockSpec((_TILE, d), lambda j: (j, 0)),
        ],
        out_specs=[
            pl.BlockSpec((1, 1), lambda j: (0, 0)),
            pl.BlockSpec((1, 1), lambda j: (0, 0)),
        ],
        out_shape=[
            jax.ShapeDtypeStruct((1, 1), jnp.float32),
            jax.ShapeDtypeStruct((1, 1), jnp.float32),
        ],
        scratch_shapes=[
            pltpu.VMEM((n, 1), jnp.float32),
            pltpu.SMEM((1, 1), jnp.float32),
        ],
        compiler_params=pltpu.CompilerParams(
            dimension_semantics=("arbitrary",)),
    )(am2, b)

    def _exact_path(am2_, b_):
        out = pl.pallas_call(
            _exact_body,
            grid=(grid,),
            in_specs=[
                pl.BlockSpec((n, d), lambda j: (0, 0)),
                pl.BlockSpec((_TILE, d), lambda j: (j, 0)),
            ],
            out_specs=pl.BlockSpec((1, 1), lambda j: (0, 0)),
            out_shape=jax.ShapeDtypeStruct((1, 1), jnp.float32),
            scratch_shapes=[
                pltpu.VMEM((n, 1), jnp.float32),
                pltpu.SMEM((1, 1), jnp.float32),
            ],
            compiler_params=pltpu.CompilerParams(
                dimension_semantics=("arbitrary",)),
        )(am2_, b_)
        return out[0, 0]

    return jax.lax.cond(
        bad[0, 0] > 0.0,
        _exact_path,
        lambda am2_, b_: loss_fast[0, 0],
        am2, b)
